# baseline (speedup 1.0000x reference)
import functools
import jax
from jax import lax
import jax.numpy as jnp
from jax.experimental import pallas as pl
from jax.experimental.pallas import tpu as pltpu
from jax.experimental.pallas import tpu_sc as plsc

N = 10000
E = 160000
BN = 1000
BE = 1000
FB = 128
NFB1 = 4

_SC_MESH = dict(core_axis_name="c", subcore_axis_name="s")
G32 = 40
NS32 = 125


def _sc_edge_gather(p, c, rows32, cols32):
    mesh = plsc.VectorSubcoreMesh(**_SC_MESH)

    @functools.partial(
        pl.kernel, mesh=mesh,
        out_type=(jax.ShapeDtypeStruct((E, 512), jnp.float32),
                  jax.ShapeDtypeStruct((E, 512), jnp.float32)),
        scratch_types=[
            pltpu.VMEM((NS32, G32), jnp.int32),
            pltpu.VMEM((NS32, G32), jnp.int32),
            pltpu.VMEM((G32, 512), jnp.float32),
            pltpu.SemaphoreType.DMA,
        ],
    )
    def k(p_h, c_h, rows_h, cols_h, gp_h, gc_h, row_v, col_v, buf_v, sem):
        cid = lax.axis_index("c")
        sid = lax.axis_index("s")
        w = cid * 16 + sid
        base = w * (E // 32)
        pltpu.sync_copy(rows_h.at[w], row_v)
        pltpu.sync_copy(cols_h.at[w], col_v)

        def body(j, carry):
            off = base + j * G32
            pltpu.async_copy(p_h.at[row_v.at[j]], buf_v, sem).wait()
            pltpu.sync_copy(buf_v, gp_h.at[pl.ds(off, G32)])
            pltpu.async_copy(c_h.at[col_v.at[j]], buf_v, sem).wait()
            pltpu.sync_copy(buf_v, gc_h.at[pl.ds(off, G32)])
            return carry
        lax.fori_loop(0, NS32, body, 0)

    return k(p, c, rows32, cols32)


def _mm_scale_body(x_ref, w_ref, dis_ref, o_ref):
    acc = jnp.dot(x_ref[...], w_ref[...], preferred_element_type=jnp.float32)
    o_ref[...] = acc * dis_ref[...]


def _mm_scale(x, w, dis):
    k = x.shape[1]
    return pl.pallas_call(
        _mm_scale_body,
        grid=(N // BN, NFB1),
        in_specs=[
            pl.BlockSpec((BN, k), lambda i, f: (i, 0)),
            pl.BlockSpec((k, FB), lambda i, f: (0, f)),
            pl.BlockSpec((BN, 1), lambda i, f: (i, 0)),
        ],
        out_specs=pl.BlockSpec((BN, FB), lambda i, f: (i, f)),
        out_shape=jax.ShapeDtypeStruct((N, 512), jnp.float32),
    )(x, w, dis)


def _post_body(relu, s_ref, y_ref, dis_ref, b_ref, o_ref):
    h = dis_ref[...] * (s_ref[...] + y_ref[...]) + b_ref[...]
    if relu:
        h = jnp.maximum(h, 0.0)
    o_ref[...] = h


def _post(s, y, dis, b, relu):
    return pl.pallas_call(
        functools.partial(_post_body, relu),
        grid=(N // BN, NFB1),
        in_specs=[
            pl.BlockSpec((BN, FB), lambda i, f: (i, f)),
            pl.BlockSpec((BN, FB), lambda i, f: (i, f)),
            pl.BlockSpec((BN, 1), lambda i, f: (i, 0)),
            pl.BlockSpec((1, FB), lambda i, f: (0, f)),
        ],
        out_specs=pl.BlockSpec((BN, FB), lambda i, f: (i, f)),
        out_shape=jax.ShapeDtypeStruct((N, 512), jnp.float32),
    )(s, y, dis, b.reshape(1, 512))


def _mm_body(x_ref, w_ref, o_ref):
    o_ref[...] = jnp.dot(x_ref[...], w_ref[...],
                         preferred_element_type=jnp.float32)


def _mm(x, w):
    return pl.pallas_call(
        _mm_body,
        grid=(N // BN, NFB1),
        in_specs=[
            pl.BlockSpec((BN, 512), lambda i, f: (i, 0)),
            pl.BlockSpec((512, FB), lambda i, f: (0, f)),
        ],
        out_specs=pl.BlockSpec((BN, FB), lambda i, f: (i, f)),
        out_shape=jax.ShapeDtypeStruct((N, 512), jnp.float32),
    )(x, w)


def _dis_body(d0_ref, d1_ref, o_ref):
    deg = 1.0 + d0_ref[:, 0:1] + d1_ref[:, 0:1]
    o_ref[...] = jax.lax.rsqrt(deg)


def _dis(d0, d1):
    return pl.pallas_call(
        _dis_body,
        grid=(N // BN,),
        in_specs=[pl.BlockSpec((BN, 16), lambda i: (i, 0)),
                  pl.BlockSpec((BN, 16), lambda i: (i, 0))],
        out_specs=pl.BlockSpec((BN, 1), lambda i: (i, 0)),
        out_shape=jax.ShapeDtypeStruct((N, 1), jnp.float32),
    )(d0, d1)


def _edge_body(gp_ref, gc_ref, wm2_ref, bm1_ref, bm2_ref, o_ref):
    z = jnp.maximum(gp_ref[...] + gc_ref[...] + bm1_ref[...], 0.0)
    o_ref[...] = jnp.dot(z, wm2_ref[...],
                         preferred_element_type=jnp.float32) + bm2_ref[...]


def _edge_mlp(gp, gc, wm2, bm1, bm2):
    return pl.pallas_call(
        _edge_body,
        grid=(E // BE,),
        in_specs=[
            pl.BlockSpec((BE, 512), lambda i: (i, 0)),
            pl.BlockSpec((BE, 512), lambda i: (i, 0)),
            pl.BlockSpec((512, 2), lambda i: (0, 0)),
            pl.BlockSpec((1, 512), lambda i: (0, 0)),
            pl.BlockSpec((1, 2), lambda i: (0, 0)),
        ],
        out_specs=pl.BlockSpec((BE, 2), lambda i: (i, 0)),
        out_shape=jax.ShapeDtypeStruct((E, 2), jnp.float32),
    )(gp, gc, wm2, bm1.reshape(1, 512), bm2.reshape(1, 2))


def kernel(x, edge_index, W1, b1, W2, b2, Wm1, bm1, Wm2, bm2):
    rows = edge_index[0].astype(jnp.int32)
    cols = edge_index[1].astype(jnp.int32)
    rows32 = rows.reshape(32, NS32, G32)
    cols32 = cols.reshape(32, NS32, G32)

    deg16 = jnp.zeros((N, 16), jnp.float32).at[cols, :].add(1.0)
    dis = _dis(deg16, jnp.zeros((N, 16), jnp.float32))

    y1 = _mm_scale(x, W1, dis)
    s1 = jnp.zeros_like(y1).at[cols].add(y1[rows])
    h = _post(s1, y1, dis, b1, relu=True)

    y2 = _mm_scale(h, W2, dis)
    s2 = jnp.zeros_like(y2).at[cols].add(y2[rows])
    h2 = _post(s2, y2, dis, b2, relu=False)

    p = _mm(h2, Wm1[:512])
    c = _mm(h2, Wm1[512:])
    gp, gc = _sc_edge_gather(p, c, rows32, cols32)
    return _edge_mlp(gp, gc, Wm2, bm1, bm2)
